# baseline (speedup 1.0000x reference)
import jax
import jax.numpy as jnp
from jax.experimental import pallas as pl
from jax.experimental.pallas import tpu as pltpu

_S = 5
_H = 24
_P = 8
_TR = 2048


def _round_up(x, m):
    return ((x + m - 1) // m) * m


def _mlp_kernel(x_ref, w1_ref, b1_ref, w2_ref, b2_ref, w3_ref, b3_ref, o_ref):
    f32 = jnp.float32

    def blockdiag(wt, rows, cols):
        tiled = jnp.concatenate([wt] * _P, axis=0)
        tiled = jnp.concatenate([tiled] * _P, axis=1)
        r = jax.lax.broadcasted_iota(jnp.int32, tiled.shape, 0) // rows
        c = jax.lax.broadcasted_iota(jnp.int32, tiled.shape, 1) // cols
        return jnp.where(r == c, tiled, 0.0)

    def bias_row(b, n):
        bt = b.T
        return jnp.concatenate([bt] * _P, axis=1)

    w1big = blockdiag(w1_ref[...].T, _S, _H)
    w2big = blockdiag(w2_ref[...].T, _H, _H)
    w3big = blockdiag(w3_ref[...].T, _H, _S)
    b1big = bias_row(b1_ref[...], _H)
    b2big = bias_row(b2_ref[...], _H)
    b3big = bias_row(b3_ref[...], _S)

    h = jnp.dot(x_ref[...], w1big, preferred_element_type=f32)
    h = jnp.maximum(h + b1big, 0.0)
    h = jnp.dot(h, w2big, preferred_element_type=f32)
    h = jnp.maximum(h + b2big, 0.0)
    h = jnp.dot(h, w3big, preferred_element_type=f32)
    o_ref[...] = jnp.maximum(h + b3big, 0.0)


def kernel(x, w1, b1, w2, b2, w3, b3):
    B = x.shape[0]
    rows_per_step = _P * _TR
    B_pad = _round_up(B, 2 * rows_per_step)
    if B_pad != B:
        x = jnp.pad(x, ((0, B_pad - B), (0, 0)))
    xv = x.reshape(B_pad // _P, _P * _S)
    num_tiles = B_pad // rows_per_step

    resident = lambda shape: pl.BlockSpec(shape, lambda i: (0, 0))
    flops = 2 * B_pad * (_S * _H + _H * _H + _H * _S)
    bytes_accessed = B_pad * (_S + _S) * 4 + 4 * (
        _S * _H + _H * _H + _S * _H + 2 * _H + _S)

    out = pl.pallas_call(
        _mlp_kernel,
        out_shape=jax.ShapeDtypeStruct((B_pad // _P, _P * _S), jnp.float32),
        grid=(num_tiles,),
        in_specs=[
            pl.BlockSpec((_TR, _P * _S), lambda i: (i, 0)),
            resident((_H, _S)),
            resident((_H, 1)),
            resident((_H, _H)),
            resident((_H, 1)),
            resident((_S, _H)),
            resident((_S, 1)),
        ],
        out_specs=pl.BlockSpec((_TR, _P * _S), lambda i: (i, 0)),
        compiler_params=pltpu.CompilerParams(
            dimension_semantics=("parallel",),
            vmem_limit_bytes=100 * 1024 * 1024,
        ),
        cost_estimate=pl.CostEstimate(
            flops=flops, transcendentals=0, bytes_accessed=bytes_accessed),
    )(xv, w1, b1, w2, b2, w3, b3)
    return out.reshape(B_pad, _S)[:B]


# speedup vs baseline: 1.0977x; 1.0977x over previous
import jax
import jax.numpy as jnp
from jax.experimental import pallas as pl
from jax.experimental.pallas import tpu as pltpu

_S = 5
_H = 24
_TB = 16384


def _round_up(x, m):
    return ((x + m - 1) // m) * m


def _mlp_kernel(x_ref, w1t_ref, b1_ref, w2_ref, b2_ref, w3_ref, b3_ref,
                o_ref, oT_ref):
    f32 = jnp.float32
    cb = _TB // 4

    def store_chunk(c):
        o_ref[c * cb:(c + 1) * cb, :] = oT_ref[:, c * cb:(c + 1) * cb].T

    store_chunk(0)
    h = jax.lax.dot_general(w1t_ref[...], x_ref[...], (((0,), (1,)), ((), ())),
                            preferred_element_type=f32)
    h = jnp.maximum(h + b1_ref[...], 0.0)
    store_chunk(1)
    h = jnp.dot(w2_ref[...], h, preferred_element_type=f32)
    h = jnp.maximum(h + b2_ref[...], 0.0)
    store_chunk(2)
    h = jnp.dot(w3_ref[...], h, preferred_element_type=f32)
    h = jnp.maximum(h + b3_ref[...], 0.0)
    store_chunk(3)
    oT_ref[...] = h


def kernel(x, w1, b1, w2, b2, w3, b3):
    B = x.shape[0]
    B_pad = _round_up(B, 2 * _TB)
    if B_pad != B:
        x = jnp.pad(x, ((0, B_pad - B), (0, 0)))
    num_tiles = B_pad // _TB
    half = num_tiles // 2

    resident = lambda shape: pl.BlockSpec(shape, lambda c, j: (0, 0))
    flops = 2 * B_pad * (_S * _H + _H * _H + _H * _S)
    bytes_accessed = B_pad * (_S + _S) * 4 + 4 * (
        _S * _H + _H * _H + _S * _H + 2 * _H + _S)

    out = pl.pallas_call(
        _mlp_kernel,
        out_shape=jax.ShapeDtypeStruct((B_pad, _S), jnp.float32),
        grid=(2, half + 1),
        in_specs=[
            pl.BlockSpec(
                (_TB, _S),
                lambda c, j: (c * half + jnp.minimum(j, half - 1), 0)),
            resident((_S, _H)),
            resident((_H, 1)),
            resident((_H, _H)),
            resident((_H, 1)),
            resident((_S, _H)),
            resident((_S, 1)),
        ],
        out_specs=pl.BlockSpec(
            (_TB, _S),
            lambda c, j: (c * half + jnp.maximum(j - 1, 0), 0)),
        scratch_shapes=[pltpu.VMEM((_S, _TB), jnp.float32)],
        compiler_params=pltpu.CompilerParams(
            dimension_semantics=("parallel", "arbitrary"),
            vmem_limit_bytes=100 * 1024 * 1024,
        ),
        cost_estimate=pl.CostEstimate(
            flops=flops, transcendentals=0, bytes_accessed=bytes_accessed),
    )(x, w1.T, b1, w2, b2, w3, b3)
    return out[:B]


# speedup vs baseline: 16.9883x; 15.4763x over previous
import jax
import jax.numpy as jnp
from jax.experimental import pallas as pl
from jax.experimental.pallas import tpu as pltpu

_S = 5
_H = 24
_TB = 32768


def _round_up(x, m):
    return ((x + m - 1) // m) * m


def _mlp_kernel(xT_ref, w1a_ref, w2a_ref, w3a_ref, o_ref):
    f32 = jnp.float32
    ones = jnp.ones((1, _TB), f32)
    xaug = jnp.concatenate([xT_ref[...], ones], axis=0)
    h = jnp.dot(w1a_ref[...], xaug, preferred_element_type=f32)
    h = jnp.maximum(h, 0.0)
    h = jnp.concatenate([h, ones], axis=0)
    h = jnp.dot(w2a_ref[...], h, preferred_element_type=f32)
    h = jnp.maximum(h, 0.0)
    h = jnp.concatenate([h, ones], axis=0)
    h = jnp.dot(w3a_ref[...], h, preferred_element_type=f32)
    o_ref[...] = jnp.maximum(h, 0.0)


def kernel(x, w1, b1, w2, b2, w3, b3):
    B = x.shape[0]
    B_pad = _round_up(B, 2 * _TB)
    xT = x.T
    if B_pad != B:
        xT = jnp.pad(xT, ((0, 0), (0, B_pad - B)))
    num_tiles = B_pad // _TB

    w1a = jnp.concatenate([w1, b1], axis=1)
    w2a = jnp.concatenate([w2, b2], axis=1)
    w3a = jnp.concatenate([w3, b3], axis=1)

    resident = lambda shape: pl.BlockSpec(shape, lambda i: (0, 0))
    flops = 2 * B_pad * (_S * _H + _H * _H + _H * _S)
    bytes_accessed = B_pad * (_S + _S) * 4 + 4 * (
        _S * _H + _H * _H + _S * _H + 2 * _H + _S)

    oT = pl.pallas_call(
        _mlp_kernel,
        out_shape=jax.ShapeDtypeStruct((_S, B_pad), jnp.float32),
        grid=(num_tiles,),
        in_specs=[
            pl.BlockSpec((_S, _TB), lambda i: (0, i)),
            resident((_H, _S + 1)),
            resident((_H, _H + 1)),
            resident((_S, _H + 1)),
        ],
        out_specs=pl.BlockSpec((_S, _TB), lambda i: (0, i)),
        compiler_params=pltpu.CompilerParams(
            dimension_semantics=("parallel",),
            vmem_limit_bytes=100 * 1024 * 1024,
        ),
        cost_estimate=pl.CostEstimate(
            flops=flops, transcendentals=0, bytes_accessed=bytes_accessed),
    )(xT, w1a, w2a, w3a)
    return oT[:, :B].T
